# baseline (speedup 1.0000x reference)
import functools

import jax
import jax.numpy as jnp
from jax import lax
from jax.experimental import pallas as pl
from jax.experimental.pallas import tpu as pltpu
from jax.experimental.pallas import tpu_sc as plsc

VOCAB = 100000
D = 64
N_HYP = 4
BATCH = 1024
SEQ = 200

C = N_HYP * D
NW = 32
CHUNK = 128
NBBLK = BATCH // CHUNK
NSLICE = 5
SSEQ = SEQ // NSLICE
UPW = SSEQ * NBBLK // NW
PAIRS = UPW // 2
CPK = C // 2


def _make_sc_kernel(s0):
    mesh = plsc.VectorSubcoreMesh(core_axis_name="c", subcore_axis_name="s")

    @functools.partial(
        pl.kernel,
        mesh=mesh,
        out_type=jax.ShapeDtypeStruct((SSEQ * BATCH, CPK), jnp.int32),
        scratch_types=[
            pltpu.VMEM((UPW * CHUNK,), jnp.int32),
            pltpu.VMEM((CHUNK, CPK), jnp.int32),
            pltpu.VMEM((CHUNK, CPK), jnp.int32),
            pltpu.SemaphoreType.DMA,
            pltpu.SemaphoreType.DMA,
            pltpu.SemaphoreType.DMA,
            pltpu.SemaphoreType.DMA,
        ],
    )
    def k(tbl_hbm, xtf_hbm, out_hbm, idx_all, rows0, rows1,
          sg0, sg1, so0, so1):
        nc = 2
        wid = lax.axis_index("s") * nc + lax.axis_index("c")
        tok0 = wid * (UPW * CHUNK)

        pltpu.sync_copy(
            xtf_hbm.at[pl.ds(s0 * BATCH + tok0, UPW * CHUNK)], idx_all)

        def idxs(u):
            return idx_all.at[pl.ds(u * CHUNK, CHUNK)]

        def dst(u):
            return out_hbm.at[pl.ds(tok0 + u * CHUNK, CHUNK)]

        pltpu.async_copy(tbl_hbm.at[idxs(0)], rows0, sg0)

        def body(i, carry):
            ua = 2 * i
            ub = ua + 1

            @pl.when(i > 0)
            def _():
                pltpu.make_async_copy(rows1, dst(ub - 2), so1).wait()

            pltpu.async_copy(tbl_hbm.at[idxs(ub)], rows1, sg1)
            pltpu.make_async_copy(tbl_hbm.at[idxs(ua)], rows0, sg0).wait()
            pltpu.async_copy(rows0, dst(ua), so0)

            @pl.when(i < PAIRS - 1)
            def _():
                pltpu.make_async_copy(rows0, dst(ua), so0).wait()
                pltpu.async_copy(tbl_hbm.at[idxs(ua + 2)], rows0, sg0)

            pltpu.make_async_copy(tbl_hbm.at[idxs(ub)], rows1, sg1).wait()
            pltpu.async_copy(rows1, dst(ub), so1)
            return carry

        lax.fori_loop(0, PAIRS, body, 0)
        pltpu.make_async_copy(rows0, dst(UPW - 2), so0).wait()
        pltpu.make_async_copy(rows1, dst(UPW - 1), so1).wait()

    return k


_sc_call = [_make_sc_kernel(i * SSEQ) for i in range(NSLICE)]


def _unpack_transpose_scale(gi, coef, out_ref):
    lo = lax.bitcast_convert_type(gi << 16, jnp.float32)
    hi = lax.bitcast_convert_type(gi & jnp.int32(-65536), jnp.float32)
    out_ref[0, 0:CPK] = lo.T * coef[0:CPK][:, None]
    out_ref[0, CPK:C] = hi.T * coef[CPK:C][:, None]


def _tc_transpose_body_first(g_ref, coef_ref, out_ref):
    _unpack_transpose_scale(g_ref[0], coef_ref[0], out_ref)


def _tc_transpose_body(carry_ref, g_ref, coef_ref, out_ref):
    del carry_ref
    _unpack_transpose_scale(g_ref[0], coef_ref[0], out_ref)


def _make_tc_transpose(s0, first):
    g_spec = pl.BlockSpec((1, BATCH, CPK), lambda j: (j, 0, 0))
    coef_spec = pl.BlockSpec((1, C), lambda j: (0, 0))
    out_spec = pl.BlockSpec((1, C, BATCH), lambda j: (s0 + j, 0, 0))
    out_shape = jax.ShapeDtypeStruct((SEQ, C, BATCH), jnp.float32)
    if first:
        return pl.pallas_call(
            _tc_transpose_body_first,
            grid=(SSEQ,),
            in_specs=[g_spec, coef_spec],
            out_specs=out_spec,
            out_shape=out_shape,
        )
    return pl.pallas_call(
        _tc_transpose_body,
        grid=(SSEQ,),
        in_specs=[pl.BlockSpec(memory_space=pl.ANY), g_spec, coef_spec],
        out_specs=out_spec,
        out_shape=out_shape,
        input_output_aliases={0: 0},
    )


_tc_transpose = [_make_tc_transpose(i * SSEQ, i == 0) for i in range(NSLICE)]


def kernel(x, tables, phases, amplitudes):
    xtf = x.T.astype(jnp.int32).reshape(SEQ * BATCH)
    t16 = tables.transpose(1, 0, 2).reshape(VOCAB, C).astype(jnp.bfloat16)
    u = lax.bitcast_convert_type(t16, jnp.uint16)
    lohi = u[:, :CPK].astype(jnp.uint32) | (u[:, CPK:].astype(jnp.uint32) << 16)
    tbl = lax.bitcast_convert_type(lohi, jnp.int32)
    coef = (jnp.cos(phases) * amplitudes[:, None]).astype(jnp.float32)
    coef = coef.reshape(1, C)

    g = [_sc_call[i](tbl, xtf) for i in range(NSLICE)]
    out = _tc_transpose[0](g[0].reshape(SSEQ, BATCH, CPK), coef)
    for i in range(1, NSLICE):
        out = _tc_transpose[i](out, g[i].reshape(SSEQ, BATCH, CPK), coef)
    out = out.reshape(SEQ, N_HYP, D, BATCH)
    return out.transpose(3, 0, 1, 2)


# speedup vs baseline: 1.2470x; 1.2470x over previous
import functools

import jax
import jax.numpy as jnp
from jax import lax
from jax.experimental import pallas as pl
from jax.experimental.pallas import tpu as pltpu
from jax.experimental.pallas import tpu_sc as plsc

VOCAB = 100000
D = 64
N_HYP = 4
BATCH = 1024
SEQ = 200

C = N_HYP * D
NW = 32
CHUNK = 128
NBBLK = BATCH // CHUNK
NSLICE = 5
SSEQ = SEQ // NSLICE
UPW = SSEQ * NBBLK // NW
PAIRS = UPW // 2
CPK = C // 2


def _make_sc_kernel(s0):
    mesh = plsc.VectorSubcoreMesh(core_axis_name="c", subcore_axis_name="s")

    @functools.partial(
        pl.kernel,
        mesh=mesh,
        out_type=jax.ShapeDtypeStruct((SSEQ * BATCH, CPK), jnp.int32),
        scratch_types=[
            pltpu.VMEM((UPW * CHUNK,), jnp.int32),
            pltpu.VMEM((CHUNK, CPK), jnp.int32),
            pltpu.VMEM((CHUNK, CPK), jnp.int32),
            pltpu.SemaphoreType.DMA,
            pltpu.SemaphoreType.DMA,
            pltpu.SemaphoreType.DMA,
            pltpu.SemaphoreType.DMA,
        ],
    )
    def k(tbl_hbm, xtf_hbm, out_hbm, idx_all, rows0, rows1,
          sg0, sg1, so0, so1):
        nc = 2
        wid = lax.axis_index("s") * nc + lax.axis_index("c")
        tok0 = wid * (UPW * CHUNK)

        pltpu.sync_copy(
            xtf_hbm.at[pl.ds(s0 * BATCH + tok0, UPW * CHUNK)], idx_all)

        def idxs(u):
            return idx_all.at[pl.ds(u * CHUNK, CHUNK)]

        def dst(u):
            return out_hbm.at[pl.ds(tok0 + u * CHUNK, CHUNK)]

        pltpu.async_copy(tbl_hbm.at[idxs(0)], rows0, sg0)

        def body(i, carry):
            ua = 2 * i
            ub = ua + 1

            @pl.when(i > 0)
            def _():
                pltpu.make_async_copy(rows1, dst(ub - 2), so1).wait()

            pltpu.async_copy(tbl_hbm.at[idxs(ub)], rows1, sg1)
            pltpu.make_async_copy(tbl_hbm.at[idxs(ua)], rows0, sg0).wait()
            pltpu.async_copy(rows0, dst(ua), so0)

            @pl.when(i < PAIRS - 1)
            def _():
                pltpu.make_async_copy(rows0, dst(ua), so0).wait()
                pltpu.async_copy(tbl_hbm.at[idxs(ua + 2)], rows0, sg0)

            pltpu.make_async_copy(tbl_hbm.at[idxs(ub)], rows1, sg1).wait()
            pltpu.async_copy(rows1, dst(ub), so1)
            return carry

        lax.fori_loop(0, PAIRS, body, 0)
        pltpu.make_async_copy(rows0, dst(UPW - 2), so0).wait()
        pltpu.make_async_copy(rows1, dst(UPW - 1), so1).wait()

    return k


_sc_call = [_make_sc_kernel(i * SSEQ) for i in range(NSLICE)]


def _unpack_transpose_scale(gi, coef, out_ref):
    lo = lax.bitcast_convert_type(gi << 16, jnp.float32)
    hi = lax.bitcast_convert_type(gi & jnp.int32(-65536), jnp.float32)
    out_ref[0, 0:CPK] = lo.T * coef[0:CPK][:, None]
    out_ref[0, CPK:C] = hi.T * coef[CPK:C][:, None]


def _tc_transpose_body_first(g_ref, coef_ref, out_ref):
    _unpack_transpose_scale(g_ref[0], coef_ref[0], out_ref)


def _tc_transpose_body(carry_ref, g_ref, coef_ref, out_ref):
    del carry_ref
    _unpack_transpose_scale(g_ref[0], coef_ref[0], out_ref)


def _make_tc_transpose(s0, first):
    g_spec = pl.BlockSpec((1, BATCH, CPK), lambda j: (j, 0, 0))
    coef_spec = pl.BlockSpec((1, C), lambda j: (0, 0))
    out_spec = pl.BlockSpec((1, C, BATCH), lambda j: (s0 + j, 0, 0))
    out_shape = jax.ShapeDtypeStruct((SEQ, C, BATCH), jnp.float32)
    if first:
        return pl.pallas_call(
            _tc_transpose_body_first,
            grid=(SSEQ,),
            in_specs=[g_spec, coef_spec],
            out_specs=out_spec,
            out_shape=out_shape,
        )
    return pl.pallas_call(
        _tc_transpose_body,
        grid=(SSEQ,),
        in_specs=[pl.BlockSpec(memory_space=pl.ANY), g_spec, coef_spec],
        out_specs=out_spec,
        out_shape=out_shape,
        input_output_aliases={0: 0},
    )


_tc_transpose = [_make_tc_transpose(i * SSEQ, i == 0) for i in range(NSLICE)]


def kernel(x, tables, phases, amplitudes):
    xtf = x.T.astype(jnp.int32).reshape(SEQ * BATCH)
    def _rne_bf16_bits(f):
        u = lax.bitcast_convert_type(f, jnp.uint32)
        return (u + 0x7FFF + ((u >> 16) & 1)) >> 16

    lo = _rne_bf16_bits(tables[0:2])
    hi = _rne_bf16_bits(tables[2:4])
    packed = lax.bitcast_convert_type(lo | (hi << 16), jnp.int32)
    tbl = packed.transpose(1, 0, 2).reshape(VOCAB, CPK)
    coef = (jnp.cos(phases) * amplitudes[:, None]).astype(jnp.float32)
    coef = coef.reshape(1, C)

    g = [_sc_call[i](tbl, xtf) for i in range(NSLICE)]
    out = _tc_transpose[0](g[0].reshape(SSEQ, BATCH, CPK), coef)
    for i in range(1, NSLICE):
        out = _tc_transpose[i](out, g[i].reshape(SSEQ, BATCH, CPK), coef)
    out = out.reshape(SEQ, N_HYP, D, BATCH)
    return out.transpose(3, 0, 1, 2)


# speedup vs baseline: 1.3104x; 1.0509x over previous
import functools

import jax
import jax.numpy as jnp
from jax import lax
from jax.experimental import pallas as pl
from jax.experimental.pallas import tpu as pltpu
from jax.experimental.pallas import tpu_sc as plsc

VOCAB = 100000
D = 64
N_HYP = 4
BATCH = 1024
SEQ = 200

C = N_HYP * D
NW = 32
CHUNK = 128
NBBLK = BATCH // CHUNK
NSLICE = 5
SSEQ = SEQ // NSLICE
UPW = SSEQ * NBBLK // NW
PAIRS = UPW // 2
CPK = C // 2


def _make_sc_kernel(s0):
    mesh = plsc.VectorSubcoreMesh(core_axis_name="c", subcore_axis_name="s")

    @functools.partial(
        pl.kernel,
        mesh=mesh,
        out_type=jax.ShapeDtypeStruct((SSEQ * BATCH, CPK), jnp.int32),
        scratch_types=[
            pltpu.VMEM((UPW * CHUNK,), jnp.int32),
            pltpu.VMEM((CHUNK, CPK), jnp.int32),
            pltpu.VMEM((CHUNK, CPK), jnp.int32),
            pltpu.SemaphoreType.DMA,
            pltpu.SemaphoreType.DMA,
            pltpu.SemaphoreType.DMA,
            pltpu.SemaphoreType.DMA,
        ],
    )
    def k(tbl_hbm, xtf_hbm, out_hbm, idx_all, rows0, rows1,
          sg0, sg1, so0, so1):
        nc = 2
        wid = lax.axis_index("s") * nc + lax.axis_index("c")
        tok0 = wid * (UPW * CHUNK)

        pltpu.sync_copy(
            xtf_hbm.at[pl.ds(s0 * BATCH + tok0, UPW * CHUNK)], idx_all)

        def idxs(u):
            return idx_all.at[pl.ds(u * CHUNK, CHUNK)]

        def dst(u):
            return out_hbm.at[pl.ds(tok0 + u * CHUNK, CHUNK)]

        pltpu.async_copy(tbl_hbm.at[idxs(0)], rows0, sg0)

        def body(i, carry):
            ua = 2 * i
            ub = ua + 1

            @pl.when(i > 0)
            def _():
                pltpu.make_async_copy(rows1, dst(ub - 2), so1).wait()

            pltpu.async_copy(tbl_hbm.at[idxs(ub)], rows1, sg1)
            pltpu.make_async_copy(tbl_hbm.at[idxs(ua)], rows0, sg0).wait()
            pltpu.async_copy(rows0, dst(ua), so0)

            @pl.when(i < PAIRS - 1)
            def _():
                pltpu.make_async_copy(rows0, dst(ua), so0).wait()
                pltpu.async_copy(tbl_hbm.at[idxs(ua + 2)], rows0, sg0)

            pltpu.make_async_copy(tbl_hbm.at[idxs(ub)], rows1, sg1).wait()
            pltpu.async_copy(rows1, dst(ub), so1)
            return carry

        lax.fori_loop(0, PAIRS, body, 0)
        pltpu.make_async_copy(rows0, dst(UPW - 2), so0).wait()
        pltpu.make_async_copy(rows1, dst(UPW - 1), so1).wait()

    return k


_sc_call = [_make_sc_kernel(i * SSEQ) for i in range(NSLICE)]


def _unpack_transpose_scale(gi, coef, out_ref):
    lo = lax.bitcast_convert_type(gi << 16, jnp.float32)
    hi = lax.bitcast_convert_type(gi & jnp.int32(-65536), jnp.float32)
    out_ref[0, 0:CPK] = lo.T * coef[0:CPK][:, None]
    out_ref[0, CPK:C] = hi.T * coef[CPK:C][:, None]


def _tc_transpose_body_first(g_ref, coef_ref, out_ref):
    _unpack_transpose_scale(g_ref[0], coef_ref[0], out_ref)


def _tc_transpose_body(carry_ref, g_ref, coef_ref, out_ref):
    del carry_ref
    _unpack_transpose_scale(g_ref[0], coef_ref[0], out_ref)


def _make_tc_transpose(s0, first):
    g_spec = pl.BlockSpec((1, BATCH, CPK), lambda j: (j, 0, 0))
    coef_spec = pl.BlockSpec((1, C), lambda j: (0, 0))
    out_spec = pl.BlockSpec((1, C, BATCH), lambda j: (s0 + j, 0, 0))
    out_shape = jax.ShapeDtypeStruct((SEQ, C, BATCH), jnp.float32)
    if first:
        return pl.pallas_call(
            _tc_transpose_body_first,
            grid=(SSEQ,),
            in_specs=[g_spec, coef_spec],
            out_specs=out_spec,
            out_shape=out_shape,
        )
    return pl.pallas_call(
        _tc_transpose_body,
        grid=(SSEQ,),
        in_specs=[pl.BlockSpec(memory_space=pl.ANY), g_spec, coef_spec],
        out_specs=out_spec,
        out_shape=out_shape,
        input_output_aliases={0: 0},
    )


_tc_transpose = [_make_tc_transpose(i * SSEQ, i == 0) for i in range(NSLICE)]

_VB = 2048


def _tbl_prep_body(t_ref, out_ref):
    u = lax.bitcast_convert_type(t_ref[...], jnp.uint32)
    r = (u + 0x7FFF + ((u >> 16) & 1)) >> 16
    w = lax.bitcast_convert_type(r[0:2] | (r[2:4] << 16), jnp.int32)
    out_ref[:, 0:D] = w[0].T
    out_ref[:, D:CPK] = w[1].T


_tbl_prep = pl.pallas_call(
    _tbl_prep_body,
    grid=((VOCAB + _VB - 1) // _VB,),
    in_specs=[pl.BlockSpec((N_HYP, D, _VB), lambda j: (0, 0, j))],
    out_specs=pl.BlockSpec((_VB, CPK), lambda j: (j, 0)),
    out_shape=jax.ShapeDtypeStruct((VOCAB, CPK), jnp.int32),
)


def kernel(x, tables, phases, amplitudes):
    xtf = x.T.astype(jnp.int32).reshape(SEQ * BATCH)
    tbl = _tbl_prep(tables.transpose(0, 2, 1))
    coef = (jnp.cos(phases) * amplitudes[:, None]).astype(jnp.float32)
    coef = coef.reshape(1, C)

    g = [_sc_call[i](tbl, xtf) for i in range(NSLICE)]
    out = _tc_transpose[0](g[0].reshape(SSEQ, BATCH, CPK), coef)
    for i in range(1, NSLICE):
        out = _tc_transpose[i](out, g[i].reshape(SSEQ, BATCH, CPK), coef)
    out = out.reshape(SEQ, N_HYP, D, BATCH)
    return out.transpose(3, 0, 1, 2)
